# Initial kernel scaffold; baseline (speedup 1.0000x reference)
import jax
import jax.numpy as jnp
from jax.experimental import pallas as pl


def kernel(x, W1, b1, W2, b2):
    raise NotImplementedError("write your pallas kernel here")


# speedup vs baseline: 4.0571x; 4.0571x over previous
import functools

import jax
import jax.numpy as jnp
from jax.experimental import pallas as pl

_N = 3136
_NP = 3200
_BLK = 128
_NBLK = _NP // _BLK
_KSEL = 17
_INF = float("inf")


def _k_select(xb_ref, xt_ref, m_ref, cs_ref):
    i = pl.program_id(0)
    xb = xb_ref[...]
    xt = xt_ref[...]
    sqb = jnp.sum(xb * xb, axis=1, keepdims=True)
    sqc = jnp.sum(xt * xt, axis=0, keepdims=True)
    prod = jax.lax.dot_general(
        xb, xt, (((1,), (0,)), ((), ())),
        preferred_element_type=jnp.float32)
    d2 = sqb + sqc - 2.0 * prod

    cols = jax.lax.broadcasted_iota(jnp.int32, (_BLK, _NP), 1)
    rows = i * _BLK + jax.lax.broadcasted_iota(jnp.int32, (_BLK, _NP), 0)
    d2 = jnp.where(cols >= _N, _INF, d2)
    d2 = jnp.where(cols == rows, -_INF, d2)

    def body(_, d):
        m = jnp.min(d, axis=1, keepdims=True)
        return jnp.where(d == m, _INF, d)

    d2 = jax.lax.fori_loop(0, _KSEL, body, d2)

    sel = (d2 == _INF) & (cols != rows) & (cols < _N) & (rows < _N)
    mb = sel.astype(jnp.float32)
    m_ref[...] = mb
    cs_ref[0, 0, :] = jnp.sum(mb, axis=0)


def _k_deg(cs_ref, dinv_ref):
    deg = 1.0 + jnp.sum(cs_ref[:, 0, :], axis=0, keepdims=True)
    dinv_ref[...] = jax.lax.rsqrt(deg)


def _k_z(hb_ref, w_ref, dinv_ref, z_ref):
    hw = jax.lax.dot_general(
        hb_ref[...], w_ref[...], (((1,), (0,)), ((), ())),
        preferred_element_type=jnp.float32)
    z_ref[...] = hw * dinv_ref[...]


def _k_agg(m_ref, z_ref, zj_ref, dinv_ref, b_ref, out_ref):
    i = pl.program_id(1)
    part = jax.lax.dot_general(
        m_ref[...], z_ref[...], (((0,), (0,)), ((), ())),
        preferred_element_type=jnp.float32)

    @pl.when(i == 0)
    def _():
        out_ref[...] = part

    @pl.when(i > 0)
    def _():
        out_ref[...] += part

    @pl.when(i == _NBLK - 1)
    def _():
        u = out_ref[...] + zj_ref[...]
        out_ref[...] = jnp.maximum(u * dinv_ref[...] + b_ref[...], 0.0)


def _gcn_layer(h, w, b, m, dinv_row, dinv_col):
    f = w.shape[1]
    z = pl.pallas_call(
        _k_z,
        grid=(_NBLK,),
        in_specs=[
            pl.BlockSpec((_BLK, h.shape[1]), lambda i: (i, 0)),
            pl.BlockSpec((h.shape[1], f), lambda i: (0, 0)),
            pl.BlockSpec((_BLK, 1), lambda i: (i, 0)),
        ],
        out_specs=pl.BlockSpec((_BLK, f), lambda i: (i, 0)),
        out_shape=jax.ShapeDtypeStruct((_NP, f), jnp.float32),
    )(h, w, dinv_col)

    out = pl.pallas_call(
        _k_agg,
        grid=(_NBLK, _NBLK),
        in_specs=[
            pl.BlockSpec((_BLK, _BLK), lambda j, i: (i, j)),
            pl.BlockSpec((_BLK, f), lambda j, i: (i, 0)),
            pl.BlockSpec((_BLK, f), lambda j, i: (j, 0)),
            pl.BlockSpec((_BLK, 1), lambda j, i: (j, 0)),
            pl.BlockSpec((1, f), lambda j, i: (0, 0)),
        ],
        out_specs=pl.BlockSpec((_BLK, f), lambda j, i: (j, 0)),
        out_shape=jax.ShapeDtypeStruct((_NP, f), jnp.float32),
    )(m, z, z, dinv_col, b.reshape(1, f))
    return out


@jax.jit
def _run(x, w1, b1, w2, b2):
    bsz, hh, ww, c = x.shape
    xf = x.reshape(bsz * hh * ww, c)
    xp = jnp.zeros((_NP, c), jnp.float32).at[:_N].set(xf)
    xt = xp.T

    m, cs = pl.pallas_call(
        _k_select,
        grid=(_NBLK,),
        in_specs=[
            pl.BlockSpec((_BLK, c), lambda i: (i, 0)),
            pl.BlockSpec((c, _NP), lambda i: (0, 0)),
        ],
        out_specs=[
            pl.BlockSpec((_BLK, _NP), lambda i: (i, 0)),
            pl.BlockSpec((1, 1, _NP), lambda i: (i, 0, 0)),
        ],
        out_shape=[
            jax.ShapeDtypeStruct((_NP, _NP), jnp.float32),
            jax.ShapeDtypeStruct((_NBLK, 1, _NP), jnp.float32),
        ],
    )(xp, xt)

    dinv_row = pl.pallas_call(
        _k_deg,
        in_specs=[pl.BlockSpec((_NBLK, 1, _NP), lambda: (0, 0, 0))],
        out_specs=pl.BlockSpec((1, _NP), lambda: (0, 0)),
        out_shape=jax.ShapeDtypeStruct((1, _NP), jnp.float32),
    )(cs)
    dinv_col = dinv_row.reshape(_NP, 1)

    h1 = _gcn_layer(xp, w1, b1, m, dinv_row, dinv_col)
    h2 = _gcn_layer(h1, w2, b2, m, dinv_row, dinv_col)
    return h2[:_N].reshape(bsz, hh, ww, -1)


def kernel(x, W1, b1, W2, b2):
    return _run(x, W1, b1, W2, b2)


# speedup vs baseline: 9.1636x; 2.2587x over previous
import jax
import jax.numpy as jnp
from jax.experimental import pallas as pl

_N = 3136
_NP = 3200
_BLK = 128
_NBLK = _NP // _BLK
_KSEL = 17
_INF = float("inf")


def _k_select(xb_ref, x_ref, mt_ref, cs_ref):
    i = pl.program_id(0)
    xb = xb_ref[...]
    x = x_ref[...]
    sqc = jnp.sum(x * x, axis=1, keepdims=True)
    prod = jax.lax.dot_general(
        x, xb, (((1,), (1,)), ((), ())),
        preferred_element_type=jnp.float32)
    e2 = sqc - 2.0 * prod

    rows = jax.lax.broadcasted_iota(jnp.int32, (_NP, _BLK), 0)
    cols = i * _BLK + jax.lax.broadcasted_iota(jnp.int32, (_NP, _BLK), 1)
    e2 = jnp.where(rows >= _N, _INF, e2)
    e2 = jnp.where(rows == cols, -_INF, e2)

    def body(_, d):
        m = jnp.min(d, axis=0, keepdims=True)
        return jnp.where(d == m, _INF, d)

    e2 = jax.lax.fori_loop(0, _KSEL, body, e2)

    sel = (e2 == _INF) & (rows != cols) & (rows < _N) & (cols < _N)
    mt_ref[...] = sel.astype(jnp.bfloat16)
    cs_ref[0] = jnp.sum(sel.astype(jnp.float32), axis=1, keepdims=True)


def _k_deg(cs_ref, dinv_ref):
    deg = 1.0 + jnp.sum(cs_ref[...], axis=0)
    dinv_ref[...] = jax.lax.rsqrt(deg)


def _k_z(hb_ref, w_ref, dinv_ref, zf_ref, zb_ref):
    hw = jax.lax.dot_general(
        hb_ref[...], w_ref[...], (((1,), (0,)), ((), ())),
        preferred_element_type=jnp.float32)
    z = hw * dinv_ref[...]
    zf_ref[...] = z
    zb_ref[...] = z.astype(jnp.bfloat16)


def _k_agg(mt_ref, zb_ref, zj_ref, dinv_ref, b_ref, out_ref):
    acc = jax.lax.dot_general(
        mt_ref[...], zb_ref[...], (((1,), (0,)), ((), ())),
        preferred_element_type=jnp.float32)
    u = acc + zj_ref[...]
    out_ref[...] = jnp.maximum(u * dinv_ref[...] + b_ref[...], 0.0)


def _gcn_layer(h, w, b, mt, dinv):
    c = h.shape[1]
    f = w.shape[1]
    zf, zb = pl.pallas_call(
        _k_z,
        grid=(_NBLK,),
        in_specs=[
            pl.BlockSpec((_BLK, c), lambda i: (i, 0)),
            pl.BlockSpec((c, f), lambda i: (0, 0)),
            pl.BlockSpec((_BLK, 1), lambda i: (i, 0)),
        ],
        out_specs=[
            pl.BlockSpec((_BLK, f), lambda i: (i, 0)),
            pl.BlockSpec((_BLK, f), lambda i: (i, 0)),
        ],
        out_shape=[
            jax.ShapeDtypeStruct((_NP, f), jnp.float32),
            jax.ShapeDtypeStruct((_NP, f), jnp.bfloat16),
        ],
    )(h, w, dinv)

    out = pl.pallas_call(
        _k_agg,
        grid=(_NBLK,),
        in_specs=[
            pl.BlockSpec((_BLK, _NP), lambda j: (j, 0)),
            pl.BlockSpec((_NP, f), lambda j: (0, 0)),
            pl.BlockSpec((_BLK, f), lambda j: (j, 0)),
            pl.BlockSpec((_BLK, 1), lambda j: (j, 0)),
            pl.BlockSpec((1, f), lambda j: (0, 0)),
        ],
        out_specs=pl.BlockSpec((_BLK, f), lambda j: (j, 0)),
        out_shape=jax.ShapeDtypeStruct((_NP, f), jnp.float32),
    )(mt, zb, zf, dinv, b.reshape(1, f))
    return out


@jax.jit
def _run(x, w1, b1, w2, b2):
    bsz, hh, ww, c = x.shape
    xf = x.reshape(bsz * hh * ww, c)
    xp = jnp.zeros((_NP, c), jnp.float32).at[:_N].set(xf)

    mt, cs = pl.pallas_call(
        _k_select,
        grid=(_NBLK,),
        in_specs=[
            pl.BlockSpec((_BLK, c), lambda i: (i, 0)),
            pl.BlockSpec((_NP, c), lambda i: (0, 0)),
        ],
        out_specs=[
            pl.BlockSpec((_NP, _BLK), lambda i: (0, i)),
            pl.BlockSpec((1, _NP, 1), lambda i: (i, 0, 0)),
        ],
        out_shape=[
            jax.ShapeDtypeStruct((_NP, _NP), jnp.bfloat16),
            jax.ShapeDtypeStruct((_NBLK, _NP, 1), jnp.float32),
        ],
    )(xp, xp)

    dinv = pl.pallas_call(
        _k_deg,
        in_specs=[pl.BlockSpec((_NBLK, _NP, 1), lambda: (0, 0, 0))],
        out_specs=pl.BlockSpec((_NP, 1), lambda: (0, 0)),
        out_shape=jax.ShapeDtypeStruct((_NP, 1), jnp.float32),
    )(cs)

    h1 = _gcn_layer(xp, w1, b1, mt, dinv)
    h2 = _gcn_layer(h1, w2, b2, mt, dinv)
    return h2[:_N].reshape(bsz, hh, ww, -1)


def kernel(x, W1, b1, W2, b2):
    return _run(x, W1, b1, W2, b2)


# speedup vs baseline: 19.9650x; 2.1787x over previous
import jax
import jax.numpy as jnp
from jax.experimental import pallas as pl

_N = 3136
_NP = 3200
_BLK = 128
_NBLK = _NP // _BLK
_KSEL = 17
_G = 32
_NG = _NP // _G
_KG = 4
_INF = float("inf")


def _k_select(xb_ref, x_ref, mt_ref, cs_ref):
    i = pl.program_id(0)
    xb = xb_ref[...]
    x = x_ref[...]
    sqc = jnp.sum(x * x, axis=1, keepdims=True)
    prod = jax.lax.dot_general(
        x, xb, (((1,), (1,)), ((), ())),
        preferred_element_type=jnp.float32)
    e2 = sqc - 2.0 * prod

    rows = jax.lax.broadcasted_iota(jnp.int32, (_NP, _BLK), 0)
    cols = i * _BLK + jax.lax.broadcasted_iota(jnp.int32, (_NP, _BLK), 1)
    e2 = jnp.where(rows >= _N, _INF, e2)
    e2 = jnp.where(rows == cols, -_INF, e2)
    valid = (rows != cols) & (rows < _N) & (cols < _N)

    e3 = e2.reshape(_NG, _G, _BLK)
    cand = []
    for t in range(_KG):
        gm = jnp.min(e3, axis=1, keepdims=True)
        cand.append(gm)
        if t < _KG - 1:
            e3 = jnp.where(e3 == gm, _INF, e3)
    tbl = jnp.concatenate(cand, axis=1).reshape(_NG * _KG, _BLK)

    def mbody(_, c):
        m = jnp.min(c, axis=0, keepdims=True)
        return jnp.where(c == m, _INF, c)

    tbl = jax.lax.fori_loop(0, _KSEL - 1, mbody, tbl)
    v17 = jnp.min(tbl, axis=0, keepdims=True)

    selall = e2 <= v17
    cnt = jnp.sum(selall.astype(jnp.float32), axis=0, keepdims=True)
    bad = jnp.any((cnt != float(_KSEL)) & (cols[:1, :] < _N))
    sel = selall & valid
    mt_ref[...] = sel.astype(jnp.bfloat16)
    cs_ref[0] = jnp.sum(sel.astype(jnp.float32), axis=1, keepdims=True)

    @pl.when(bad)
    def _():
        def body(_, d):
            m = jnp.min(d, axis=0, keepdims=True)
            return jnp.where(d == m, _INF, d)

        d = jax.lax.fori_loop(0, _KSEL, body, e2)
        s = (d == _INF) & valid
        mt_ref[...] = s.astype(jnp.bfloat16)
        cs_ref[0] = jnp.sum(s.astype(jnp.float32), axis=1, keepdims=True)


def _k_deg(cs_ref, dinv_ref):
    deg = 1.0 + jnp.sum(cs_ref[...], axis=0)
    dinv_ref[...] = jax.lax.rsqrt(deg)


def _k_z(hb_ref, w_ref, dinv_ref, zf_ref, zb_ref):
    hw = jax.lax.dot_general(
        hb_ref[...], w_ref[...], (((1,), (0,)), ((), ())),
        preferred_element_type=jnp.float32)
    z = hw * dinv_ref[...]
    zf_ref[...] = z
    zb_ref[...] = z.astype(jnp.bfloat16)


def _k_agg(mt_ref, zb_ref, zj_ref, dinv_ref, b_ref, out_ref):
    acc = jax.lax.dot_general(
        mt_ref[...], zb_ref[...], (((1,), (0,)), ((), ())),
        preferred_element_type=jnp.float32)
    u = acc + zj_ref[...]
    out_ref[...] = jnp.maximum(u * dinv_ref[...] + b_ref[...], 0.0)


def _gcn_layer(h, w, b, mt, dinv):
    c = h.shape[1]
    f = w.shape[1]
    zf, zb = pl.pallas_call(
        _k_z,
        grid=(_NBLK,),
        in_specs=[
            pl.BlockSpec((_BLK, c), lambda i: (i, 0)),
            pl.BlockSpec((c, f), lambda i: (0, 0)),
            pl.BlockSpec((_BLK, 1), lambda i: (i, 0)),
        ],
        out_specs=[
            pl.BlockSpec((_BLK, f), lambda i: (i, 0)),
            pl.BlockSpec((_BLK, f), lambda i: (i, 0)),
        ],
        out_shape=[
            jax.ShapeDtypeStruct((_NP, f), jnp.float32),
            jax.ShapeDtypeStruct((_NP, f), jnp.bfloat16),
        ],
    )(h, w, dinv)

    out = pl.pallas_call(
        _k_agg,
        grid=(_NBLK,),
        in_specs=[
            pl.BlockSpec((_BLK, _NP), lambda j: (j, 0)),
            pl.BlockSpec((_NP, f), lambda j: (0, 0)),
            pl.BlockSpec((_BLK, f), lambda j: (j, 0)),
            pl.BlockSpec((_BLK, 1), lambda j: (j, 0)),
            pl.BlockSpec((1, f), lambda j: (0, 0)),
        ],
        out_specs=pl.BlockSpec((_BLK, f), lambda j: (j, 0)),
        out_shape=jax.ShapeDtypeStruct((_NP, f), jnp.float32),
    )(mt, zb, zf, dinv, b.reshape(1, f))
    return out


@jax.jit
def _run(x, w1, b1, w2, b2):
    bsz, hh, ww, c = x.shape
    xf = x.reshape(bsz * hh * ww, c)
    xp = jnp.zeros((_NP, c), jnp.float32).at[:_N].set(xf)

    mt, cs = pl.pallas_call(
        _k_select,
        grid=(_NBLK,),
        in_specs=[
            pl.BlockSpec((_BLK, c), lambda i: (i, 0)),
            pl.BlockSpec((_NP, c), lambda i: (0, 0)),
        ],
        out_specs=[
            pl.BlockSpec((_NP, _BLK), lambda i: (0, i)),
            pl.BlockSpec((1, _NP, 1), lambda i: (i, 0, 0)),
        ],
        out_shape=[
            jax.ShapeDtypeStruct((_NP, _NP), jnp.bfloat16),
            jax.ShapeDtypeStruct((_NBLK, _NP, 1), jnp.float32),
        ],
    )(xp, xp)

    dinv = pl.pallas_call(
        _k_deg,
        in_specs=[pl.BlockSpec((_NBLK, _NP, 1), lambda: (0, 0, 0))],
        out_specs=pl.BlockSpec((_NP, 1), lambda: (0, 0)),
        out_shape=jax.ShapeDtypeStruct((_NP, 1), jnp.float32),
    )(cs)

    h1 = _gcn_layer(xp, w1, b1, mt, dinv)
    h2 = _gcn_layer(h1, w2, b2, mt, dinv)
    return h2[:_N].reshape(bsz, hh, ww, -1)


def kernel(x, W1, b1, W2, b2):
    return _run(x, W1, b1, W2, b2)


# speedup vs baseline: 22.2702x; 1.1155x over previous
import jax
import jax.numpy as jnp
from jax.experimental import pallas as pl

_N = 3136
_NP = 3200
_BLK = 128
_NBLK = _NP // _BLK
_KSEL = 17
_G = 32
_NG = _NP // _G
_KG = 4
_INF = float("inf")


def _k_select(xb_ref, x_ref, w1_ref, mt_ref, cs_ref, xw_ref):
    i = pl.program_id(0)
    xb = xb_ref[...]
    x = x_ref[...]
    xw_ref[...] = jax.lax.dot_general(
        xb, w1_ref[...], (((1,), (0,)), ((), ())),
        preferred_element_type=jnp.float32)
    sqc = jnp.sum(x * x, axis=1, keepdims=True)
    prod = jax.lax.dot_general(
        x, xb, (((1,), (1,)), ((), ())),
        preferred_element_type=jnp.float32)
    e2 = sqc - 2.0 * prod

    rows = jax.lax.broadcasted_iota(jnp.int32, (_NP, _BLK), 0)
    cols = i * _BLK + jax.lax.broadcasted_iota(jnp.int32, (_NP, _BLK), 1)
    e2 = jnp.where(rows >= _N, _INF, e2)
    e2 = jnp.where(rows == cols, -_INF, e2)
    valid = (rows != cols) & (rows < _N) & (cols < _N)

    e3 = e2.reshape(_NG, _G, _BLK)
    cand = []
    for t in range(_KG):
        gm = jnp.min(e3, axis=1, keepdims=True)
        cand.append(gm)
        if t < _KG - 1:
            e3 = jnp.where(e3 == gm, _INF, e3)
    tbl = jnp.concatenate(cand, axis=1).reshape(_NG * _KG, _BLK)

    def mbody(_, c):
        m = jnp.min(c, axis=0, keepdims=True)
        return jnp.where(c == m, _INF, c)

    tbl = jax.lax.fori_loop(0, _KSEL - 1, mbody, tbl)
    v17 = jnp.min(tbl, axis=0, keepdims=True)

    selall = e2 <= v17
    cnt = jnp.sum(selall.astype(jnp.float32), axis=0, keepdims=True)
    bad = jnp.any((cnt != float(_KSEL)) & (cols[:1, :] < _N))
    sel = selall & valid
    mt_ref[...] = sel.astype(jnp.bfloat16)
    cs_ref[0] = jnp.sum(sel.astype(jnp.float32), axis=1, keepdims=True)

    @pl.when(bad)
    def _():
        def body(_, d):
            m = jnp.min(d, axis=0, keepdims=True)
            return jnp.where(d == m, _INF, d)

        d = jax.lax.fori_loop(0, _KSEL, body, e2)
        s = (d == _INF) & valid
        mt_ref[...] = s.astype(jnp.bfloat16)
        cs_ref[0] = jnp.sum(s.astype(jnp.float32), axis=1, keepdims=True)


def _k_degz(cs_ref, xw_ref, dinv_ref, zf_ref, zb_ref):
    deg = 1.0 + jnp.sum(cs_ref[...], axis=0)
    dinv = jax.lax.rsqrt(deg)
    dinv_ref[...] = dinv
    z = xw_ref[...] * dinv
    zf_ref[...] = z
    zb_ref[...] = z.astype(jnp.bfloat16)


def _k_aggz(mt_ref, zb_ref, zj_ref, dinv_ref, b_ref, w2_ref,
            z2f_ref, z2b_ref):
    acc = jax.lax.dot_general(
        mt_ref[...], zb_ref[...], (((1,), (0,)), ((), ())),
        preferred_element_type=jnp.float32)
    dinv = dinv_ref[...]
    h1 = jnp.maximum((acc + zj_ref[...]) * dinv + b_ref[...], 0.0)
    hw = jax.lax.dot_general(
        h1, w2_ref[...], (((1,), (0,)), ((), ())),
        preferred_element_type=jnp.float32)
    z2 = hw * dinv
    z2f_ref[...] = z2
    z2b_ref[...] = z2.astype(jnp.bfloat16)


def _k_agg(mt_ref, zb_ref, zj_ref, dinv_ref, b_ref, out_ref):
    acc = jax.lax.dot_general(
        mt_ref[...], zb_ref[...], (((1,), (0,)), ((), ())),
        preferred_element_type=jnp.float32)
    u = acc + zj_ref[...]
    out_ref[...] = jnp.maximum(u * dinv_ref[...] + b_ref[...], 0.0)


@jax.jit
def _run(x, w1, b1, w2, b2):
    bsz, hh, ww, c = x.shape
    f = w1.shape[1]
    xf = x.reshape(bsz * hh * ww, c)
    xp = jnp.zeros((_NP, c), jnp.float32).at[:_N].set(xf)

    mt, cs, xw1 = pl.pallas_call(
        _k_select,
        grid=(_NBLK,),
        in_specs=[
            pl.BlockSpec((_BLK, c), lambda i: (i, 0)),
            pl.BlockSpec((_NP, c), lambda i: (0, 0)),
            pl.BlockSpec((c, f), lambda i: (0, 0)),
        ],
        out_specs=[
            pl.BlockSpec((_NP, _BLK), lambda i: (0, i)),
            pl.BlockSpec((1, _NP, 1), lambda i: (i, 0, 0)),
            pl.BlockSpec((_BLK, f), lambda i: (i, 0)),
        ],
        out_shape=[
            jax.ShapeDtypeStruct((_NP, _NP), jnp.bfloat16),
            jax.ShapeDtypeStruct((_NBLK, _NP, 1), jnp.float32),
            jax.ShapeDtypeStruct((_NP, f), jnp.float32),
        ],
    )(xp, xp, w1)

    dinv, z1f, z1b = pl.pallas_call(
        _k_degz,
        grid=(_NBLK,),
        in_specs=[
            pl.BlockSpec((_NBLK, _BLK, 1), lambda j: (0, j, 0)),
            pl.BlockSpec((_BLK, f), lambda j: (j, 0)),
        ],
        out_specs=[
            pl.BlockSpec((_BLK, 1), lambda j: (j, 0)),
            pl.BlockSpec((_BLK, f), lambda j: (j, 0)),
            pl.BlockSpec((_BLK, f), lambda j: (j, 0)),
        ],
        out_shape=[
            jax.ShapeDtypeStruct((_NP, 1), jnp.float32),
            jax.ShapeDtypeStruct((_NP, f), jnp.float32),
            jax.ShapeDtypeStruct((_NP, f), jnp.bfloat16),
        ],
    )(cs, xw1)

    z2f, z2b = pl.pallas_call(
        _k_aggz,
        grid=(_NBLK,),
        in_specs=[
            pl.BlockSpec((_BLK, _NP), lambda j: (j, 0)),
            pl.BlockSpec((_NP, f), lambda j: (0, 0)),
            pl.BlockSpec((_BLK, f), lambda j: (j, 0)),
            pl.BlockSpec((_BLK, 1), lambda j: (j, 0)),
            pl.BlockSpec((1, f), lambda j: (0, 0)),
            pl.BlockSpec((f, f), lambda j: (0, 0)),
        ],
        out_specs=[
            pl.BlockSpec((_BLK, f), lambda j: (j, 0)),
            pl.BlockSpec((_BLK, f), lambda j: (j, 0)),
        ],
        out_shape=[
            jax.ShapeDtypeStruct((_NP, f), jnp.float32),
            jax.ShapeDtypeStruct((_NP, f), jnp.bfloat16),
        ],
    )(mt, z1b, z1f, dinv, b1.reshape(1, f), w2)

    h2 = pl.pallas_call(
        _k_agg,
        grid=(_NBLK,),
        in_specs=[
            pl.BlockSpec((_BLK, _NP), lambda j: (j, 0)),
            pl.BlockSpec((_NP, f), lambda j: (0, 0)),
            pl.BlockSpec((_BLK, f), lambda j: (j, 0)),
            pl.BlockSpec((_BLK, 1), lambda j: (j, 0)),
            pl.BlockSpec((1, f), lambda j: (0, 0)),
        ],
        out_specs=pl.BlockSpec((_BLK, f), lambda j: (j, 0)),
        out_shape=jax.ShapeDtypeStruct((_NP, f), jnp.float32),
    )(mt, z2b, z2f, dinv, b2.reshape(1, f))
    return h2[:_N].reshape(bsz, hh, ww, -1)


def kernel(x, W1, b1, W2, b2):
    return _run(x, W1, b1, W2, b2)


# speedup vs baseline: 24.6752x; 1.1080x over previous
import jax
import jax.numpy as jnp
from jax.experimental import pallas as pl
from jax.experimental.pallas import tpu as pltpu

_N = 3136
_NP = 3200
_BLK = 128
_NBLK = _NP // _BLK
_KSEL = 17
_G = 32
_NG = _NP // _G
_KG = 4
_INF = float("inf")


def _k_select(xb_ref, x_ref, w1_ref, mt_ref, dinv_ref, zf_ref, zb_ref,
              cs_scr, xw_scr):
    i = pl.program_id(0)
    xb = xb_ref[...]
    x = x_ref[...]
    xw_scr[pl.ds(i * _BLK, _BLK), :] = jax.lax.dot_general(
        xb, w1_ref[...], (((1,), (0,)), ((), ())),
        preferred_element_type=jnp.float32)
    sqc = jnp.sum(x * x, axis=1, keepdims=True)
    prod = jax.lax.dot_general(
        x, xb, (((1,), (1,)), ((), ())),
        preferred_element_type=jnp.float32)
    e2 = sqc - 2.0 * prod

    rows = jax.lax.broadcasted_iota(jnp.int32, (_NP, _BLK), 0)
    cols = i * _BLK + jax.lax.broadcasted_iota(jnp.int32, (_NP, _BLK), 1)
    e2 = jnp.where(rows >= _N, _INF, e2)
    e2 = jnp.where(rows == cols, -_INF, e2)
    valid = (rows != cols) & (rows < _N) & (cols < _N)

    e3 = e2.reshape(_NG, _G, _BLK)
    cand = []
    for t in range(_KG):
        gm = jnp.min(e3, axis=1, keepdims=True)
        cand.append(gm)
        if t < _KG - 1:
            e3 = jnp.where(e3 == gm, _INF, e3)
    tbl = jnp.concatenate(cand, axis=1).reshape(_NG * _KG, _BLK)

    def mbody(_, c):
        m = jnp.min(c, axis=0, keepdims=True)
        return jnp.where(c == m, _INF, c)

    tbl = jax.lax.fori_loop(0, _KSEL - 1, mbody, tbl)
    v17 = jnp.min(tbl, axis=0, keepdims=True)

    selall = e2 <= v17
    cnt = jnp.sum(selall.astype(jnp.float32), axis=0, keepdims=True)
    bad = jnp.any((cnt != float(_KSEL)) & (cols[:1, :] < _N))
    sel = selall & valid
    mt_ref[...] = sel.astype(jnp.bfloat16)
    cs_scr[i] = jnp.sum(sel.astype(jnp.float32), axis=1, keepdims=True)

    @pl.when(bad)
    def _():
        def body(_, d):
            m = jnp.min(d, axis=0, keepdims=True)
            return jnp.where(d == m, _INF, d)

        d = jax.lax.fori_loop(0, _KSEL, body, e2)
        s = (d == _INF) & valid
        mt_ref[...] = s.astype(jnp.bfloat16)
        cs_scr[i] = jnp.sum(s.astype(jnp.float32), axis=1, keepdims=True)

    @pl.when(i == _NBLK - 1)
    def _():
        deg = 1.0 + jnp.sum(cs_scr[...], axis=0)
        dinv = jax.lax.rsqrt(deg)
        dinv_ref[...] = dinv
        z = xw_scr[...] * dinv
        zf_ref[...] = z
        zb_ref[...] = z.astype(jnp.bfloat16)


def _k_aggz(mt_ref, zb_ref, zj_ref, dinv_ref, b_ref, w2_ref,
            z2f_ref, z2b_ref):
    acc = jax.lax.dot_general(
        mt_ref[...], zb_ref[...], (((1,), (0,)), ((), ())),
        preferred_element_type=jnp.float32)
    dinv = dinv_ref[...]
    h1 = jnp.maximum((acc + zj_ref[...]) * dinv + b_ref[...], 0.0)
    hw = jax.lax.dot_general(
        h1, w2_ref[...], (((1,), (0,)), ((), ())),
        preferred_element_type=jnp.float32)
    z2 = hw * dinv
    z2f_ref[...] = z2
    z2b_ref[...] = z2.astype(jnp.bfloat16)


def _k_agg(mt_ref, zb_ref, zj_ref, dinv_ref, b_ref, out_ref):
    acc = jax.lax.dot_general(
        mt_ref[...], zb_ref[...], (((1,), (0,)), ((), ())),
        preferred_element_type=jnp.float32)
    u = acc + zj_ref[...]
    out_ref[...] = jnp.maximum(u * dinv_ref[...] + b_ref[...], 0.0)


@jax.jit
def _run(x, w1, b1, w2, b2):
    bsz, hh, ww, c = x.shape
    f = w1.shape[1]
    xf = x.reshape(bsz * hh * ww, c)
    xp = jnp.zeros((_NP, c), jnp.float32).at[:_N].set(xf)

    mt, dinv, z1f, z1b = pl.pallas_call(
        _k_select,
        grid=(_NBLK,),
        in_specs=[
            pl.BlockSpec((_BLK, c), lambda i: (i, 0)),
            pl.BlockSpec((_NP, c), lambda i: (0, 0)),
            pl.BlockSpec((c, f), lambda i: (0, 0)),
        ],
        out_specs=[
            pl.BlockSpec((_NP, _BLK), lambda i: (0, i)),
            pl.BlockSpec((_NP, 1), lambda i: (0, 0)),
            pl.BlockSpec((_NP, f), lambda i: (0, 0)),
            pl.BlockSpec((_NP, f), lambda i: (0, 0)),
        ],
        out_shape=[
            jax.ShapeDtypeStruct((_NP, _NP), jnp.bfloat16),
            jax.ShapeDtypeStruct((_NP, 1), jnp.float32),
            jax.ShapeDtypeStruct((_NP, f), jnp.float32),
            jax.ShapeDtypeStruct((_NP, f), jnp.bfloat16),
        ],
        scratch_shapes=[
            pltpu.VMEM((_NBLK, _NP, 1), jnp.float32),
            pltpu.VMEM((_NP, 256), jnp.float32),
        ],
        compiler_params=pltpu.CompilerParams(
            vmem_limit_bytes=100 * 1024 * 1024),
    )(xp, xp, w1)

    z2f, z2b = pl.pallas_call(
        _k_aggz,
        grid=(_NBLK,),
        in_specs=[
            pl.BlockSpec((_BLK, _NP), lambda j: (j, 0)),
            pl.BlockSpec((_NP, f), lambda j: (0, 0)),
            pl.BlockSpec((_BLK, f), lambda j: (j, 0)),
            pl.BlockSpec((_BLK, 1), lambda j: (j, 0)),
            pl.BlockSpec((1, f), lambda j: (0, 0)),
            pl.BlockSpec((f, f), lambda j: (0, 0)),
        ],
        out_specs=[
            pl.BlockSpec((_BLK, f), lambda j: (j, 0)),
            pl.BlockSpec((_BLK, f), lambda j: (j, 0)),
        ],
        out_shape=[
            jax.ShapeDtypeStruct((_NP, f), jnp.float32),
            jax.ShapeDtypeStruct((_NP, f), jnp.bfloat16),
        ],
    )(mt, z1b, z1f, dinv, b1.reshape(1, f), w2)

    h2 = pl.pallas_call(
        _k_agg,
        grid=(_NBLK,),
        in_specs=[
            pl.BlockSpec((_BLK, _NP), lambda j: (j, 0)),
            pl.BlockSpec((_NP, f), lambda j: (0, 0)),
            pl.BlockSpec((_BLK, f), lambda j: (j, 0)),
            pl.BlockSpec((_BLK, 1), lambda j: (j, 0)),
            pl.BlockSpec((1, f), lambda j: (0, 0)),
        ],
        out_specs=pl.BlockSpec((_BLK, f), lambda j: (j, 0)),
        out_shape=jax.ShapeDtypeStruct((_NP, f), jnp.float32),
    )(mt, z2b, z2f, dinv, b2.reshape(1, f))
    return h2[:_N].reshape(bsz, hh, ww, -1)


def kernel(x, W1, b1, W2, b2):
    return _run(x, W1, b1, W2, b2)


# speedup vs baseline: 25.1835x; 1.0206x over previous
import jax
import jax.numpy as jnp
from jax.experimental import pallas as pl
from jax.experimental.pallas import tpu as pltpu

_N = 3136
_NP = 3200
_BLK = 128
_NBLK = _NP // _BLK
_F = 256
_KSEL = 17
_G = 32
_NG = _NP // _G
_KG = 4
_INF = float("inf")


def _k_all(xb_ref, x_ref, w1_ref, w2_ref, b1_ref, b2_ref, out_ref,
           mt_scr, xw_scr, dinv_scr,
           z1f_scr, z1b_scr, z2f_scr, z2b_scr):
    p = pl.program_id(0)

    @pl.when(p < _NBLK)
    def _phase_select():
        i = p
        xb = xb_ref[...]
        x = x_ref[...]
        xw_scr[pl.ds(i * _BLK, _BLK), :] = jax.lax.dot_general(
            xb, w1_ref[...], (((1,), (0,)), ((), ())),
            preferred_element_type=jnp.float32)
        sqc = jnp.sum(x * x, axis=1, keepdims=True)
        prod = jax.lax.dot_general(
            x, xb, (((1,), (1,)), ((), ())),
            preferred_element_type=jnp.float32)
        e2 = sqc - 2.0 * prod

        rows = jax.lax.broadcasted_iota(jnp.int32, (_NP, _BLK), 0)
        cols = i * _BLK + jax.lax.broadcasted_iota(jnp.int32, (_NP, _BLK), 1)
        e2 = jnp.where(rows >= _N, _INF, e2)
        e2 = jnp.where(rows == cols, -_INF, e2)
        valid = (rows != cols) & (rows < _N) & (cols < _N)

        e3 = e2.reshape(_NG, _G, _BLK)
        cand = []
        for t in range(_KG):
            gm = jnp.min(e3, axis=1, keepdims=True)
            cand.append(gm)
            if t < _KG - 1:
                e3 = jnp.where(e3 == gm, _INF, e3)
        tbl = jnp.concatenate(cand, axis=1).reshape(_NG * _KG, _BLK)

        def mbody(_, c):
            m = jnp.min(c, axis=0, keepdims=True)
            return jnp.where(c == m, _INF, c)

        tbl = jax.lax.fori_loop(0, _KSEL - 1, mbody, tbl)
        v17 = jnp.min(tbl, axis=0, keepdims=True)

        selall = e2 <= v17
        cnt = jnp.sum(selall.astype(jnp.float32), axis=0, keepdims=True)
        bad = jnp.any((cnt != float(_KSEL)) & (cols[:1, :] < _N))
        sel = selall & valid
        mt_scr[i] = sel.astype(jnp.bfloat16)

        @pl.when(bad)
        def _():
            def body(_, d):
                m = jnp.min(d, axis=0, keepdims=True)
                return jnp.where(d == m, _INF, d)

            d = jax.lax.fori_loop(0, _KSEL, body, e2)
            s = (d == _INF) & valid
            mt_scr[i] = s.astype(jnp.bfloat16)

        @pl.when(i == _NBLK - 1)
        def _():
            deg = jnp.ones((_NP, 1), jnp.float32)
            for k in range(_NBLK):
                deg += jnp.sum(mt_scr[k].astype(jnp.float32),
                               axis=1, keepdims=True)
            dinv = jax.lax.rsqrt(deg)
            dinv_scr[...] = dinv
            z = xw_scr[...] * dinv
            z1f_scr[...] = z
            z1b_scr[...] = z.astype(jnp.bfloat16)

    def _mt_matmul(j, zb_scr):
        jds = pl.ds(j * _BLK, _BLK)
        acc = jnp.zeros((_BLK, _F), jnp.float32)
        for k in range(_NBLK):
            acc += jax.lax.dot_general(
                mt_scr[k, jds, :], zb_scr[pl.ds(k * _BLK, _BLK), :],
                (((1,), (0,)), ((), ())),
                preferred_element_type=jnp.float32)
        return acc, jds

    @pl.when((p >= _NBLK) & (p < 2 * _NBLK))
    def _phase_layer1():
        j = p - _NBLK
        acc, jds = _mt_matmul(j, z1b_scr)
        dinv = dinv_scr[jds, :]
        h1 = jnp.maximum((acc + z1f_scr[jds, :]) * dinv + b1_ref[...], 0.0)
        z2 = jax.lax.dot_general(
            h1, w2_ref[...], (((1,), (0,)), ((), ())),
            preferred_element_type=jnp.float32) * dinv
        z2f_scr[jds, :] = z2
        z2b_scr[jds, :] = z2.astype(jnp.bfloat16)

    @pl.when(p >= 2 * _NBLK)
    def _phase_layer2():
        j = p - 2 * _NBLK
        acc, jds = _mt_matmul(j, z2b_scr)
        u = acc + z2f_scr[jds, :]
        out_ref[...] = jnp.maximum(u * dinv_scr[jds, :] + b2_ref[...], 0.0)


@jax.jit
def _run(x, w1, b1, w2, b2):
    bsz, hh, ww, c = x.shape
    f = w1.shape[1]
    xf = x.reshape(bsz * hh * ww, c)
    xp = jnp.zeros((_NP, c), jnp.float32).at[:_N].set(xf)

    h2 = pl.pallas_call(
        _k_all,
        grid=(3 * _NBLK,),
        in_specs=[
            pl.BlockSpec((_BLK, c), lambda p: (jnp.where(p < _NBLK, p, 0), 0)),
            pl.BlockSpec((_NP, c), lambda p: (0, 0)),
            pl.BlockSpec((c, f), lambda p: (0, 0)),
            pl.BlockSpec((f, f), lambda p: (0, 0)),
            pl.BlockSpec((1, f), lambda p: (0, 0)),
            pl.BlockSpec((1, f), lambda p: (0, 0)),
        ],
        out_specs=pl.BlockSpec(
            (_BLK, f), lambda p: (jnp.maximum(p - 2 * _NBLK, 0), 0)),
        out_shape=jax.ShapeDtypeStruct((_NP, f), jnp.float32),
        scratch_shapes=[
            pltpu.VMEM((_NBLK, _NP, _BLK), jnp.bfloat16),
            pltpu.VMEM((_NP, _F), jnp.float32),
            pltpu.VMEM((_NP, 1), jnp.float32),
            pltpu.VMEM((_NP, _F), jnp.float32),
            pltpu.VMEM((_NP, _F), jnp.bfloat16),
            pltpu.VMEM((_NP, _F), jnp.float32),
            pltpu.VMEM((_NP, _F), jnp.bfloat16),
        ],
        compiler_params=pltpu.CompilerParams(
            vmem_limit_bytes=100 * 1024 * 1024),
    )(xp, xp, w1, w2, b1.reshape(1, f), b2.reshape(1, f))
    return h2[:_N].reshape(bsz, hh, ww, -1)


def kernel(x, W1, b1, W2, b2):
    return _run(x, W1, b1, W2, b2)
